# baseline (speedup 1.0000x reference)
import functools

import jax
import jax.numpy as jnp
from jax import lax
from jax.experimental import pallas as pl
from jax.experimental.pallas import tpu as pltpu
from jax.experimental.pallas import tpu_sc as plsc

N = 10000
E = 320000
D = 128
F = 32
FP = 128

NC = 2
NS = 16
NW = NC * NS

CH = 128
NCH = 80
EPW = NCH * CH
EP = NW * EPW
NP = 10240
RT = NP // NS

_mesh = plsc.VectorSubcoreMesh(core_axis_name="c", subcore_axis_name="s")

HIGHEST = lax.Precision.HIGHEST


@functools.partial(
    pl.kernel,
    out_type=jax.ShapeDtypeStruct((NC, NP), jnp.float32),
    mesh=_mesh,
    scratch_types=[
        pltpu.VMEM((NCH, CH), jnp.int32),
        pltpu.VMEM((CH,), jnp.float32),
        pltpu.VMEM_SHARED((NP,), jnp.float32),
        pltpu.SemaphoreType.DMA,
    ],
)
def _sc_degree(dstw_hbm, zn_hbm, ones_hbm, out_hbm, dst_v, ones_v, acc_sh, sem):
    c = lax.axis_index("c")
    s = lax.axis_index("s")
    wid = s * NC + c
    r0 = s * RT
    pltpu.sync_copy(zn_hbm.at[pl.ds(r0, RT)], acc_sh.at[pl.ds(r0, RT)])
    pltpu.sync_copy(dstw_hbm.at[wid], dst_v)
    pltpu.sync_copy(ones_hbm, ones_v)
    plsc.subcore_barrier()

    def body(j, carry):
        pltpu.sync_copy(ones_v, acc_sh.at[dst_v.at[j]], add=True)
        return carry

    lax.fori_loop(0, NCH, body, 0, unroll=False)
    plsc.subcore_barrier()
    pltpu.sync_copy(acc_sh.at[pl.ds(r0, RT)], out_hbm.at[c, pl.ds(r0, RT)])


@functools.partial(
    pl.kernel,
    out_type=jax.ShapeDtypeStruct((NC, NP, FP), jnp.float32),
    mesh=_mesh,
    scratch_types=[
        pltpu.VMEM((NCH, CH), jnp.int32),
        pltpu.VMEM((CH,), jnp.int32),
        pltpu.VMEM((CH,), jnp.int32),
        pltpu.VMEM((CH, FP), jnp.float32),
        pltpu.VMEM((CH, FP), jnp.float32),
        pltpu.VMEM_SHARED((NP, FP), jnp.float32),
        pltpu.SemaphoreType.DMA,
        pltpu.SemaphoreType.DMA,
        pltpu.SemaphoreType.DMA,
        pltpu.SemaphoreType.DMA,
    ],
)
def _sc_scatter(q_hbm, srcw_hbm, dstw_hbm, znf_hbm, out_hbm,
                src_v, dst0, dst1, rows0, rows1, acc_sh,
                semg0, semg1, sems0, sems1):
    c = lax.axis_index("c")
    s = lax.axis_index("s")
    wid = s * NC + c
    r0 = s * RT
    pltpu.sync_copy(znf_hbm.at[pl.ds(r0, RT)], acc_sh.at[pl.ds(r0, RT)])
    pltpu.sync_copy(srcw_hbm.at[wid], src_v)
    plsc.subcore_barrier()

    def gstart(j, rows_v, dst_v, semg):
        pltpu.async_copy(q_hbm.at[src_v.at[j]], rows_v, semg)
        pltpu.async_copy(dstw_hbm.at[wid, j], dst_v, semg)

    def gwait_sstart(j, rows_v, dst_v, semg, sems):
        pltpu.make_async_copy(q_hbm.at[src_v.at[j]], rows_v, semg).wait()
        pltpu.make_async_copy(dstw_hbm.at[wid, j], dst_v, semg).wait()
        pltpu.async_copy(rows_v, acc_sh.at[dst_v], sems, add=True)

    def swait(rows_v, dst_v, sems):
        pltpu.make_async_copy(rows_v, acc_sh.at[dst_v], sems).wait()

    gstart(0, rows0, dst0, semg0)
    gstart(1, rows1, dst1, semg1)

    def body(i, carry):
        j = 2 * i
        gwait_sstart(j, rows0, dst0, semg0, sems0)
        gwait_sstart(j + 1, rows1, dst1, semg1, sems1)
        swait(rows0, dst0, sems0)
        gstart(j + 2, rows0, dst0, semg0)
        swait(rows1, dst1, sems1)
        gstart(j + 3, rows1, dst1, semg1)
        return carry

    lax.fori_loop(0, NCH // 2 - 1, body, 0, unroll=False)
    gwait_sstart(NCH - 2, rows0, dst0, semg0, sems0)
    gwait_sstart(NCH - 1, rows1, dst1, semg1, sems1)
    swait(rows0, dst0, sems0)
    swait(rows1, dst1, sems1)
    plsc.subcore_barrier()
    pltpu.sync_copy(acc_sh.at[pl.ds(r0, RT)], out_hbm.at[c, pl.ds(r0, RT)])


def _tc_a_body(degp_ref, x_ref, w1_ref, q_ref):
    degp = degp_ref[...]
    deg = degp[0, :N] + degp[1, :N] + 1.0
    dinv = lax.rsqrt(deg)
    p = jnp.dot(x_ref[...], w1_ref[...],
                preferred_element_type=jnp.float32, precision=HIGHEST)
    q_ref[...] = jnp.pad(p * dinv[:, None], ((0, NP - N), (0, FP - F)))


def _tc_b_body(aggp_ref, q1_ref, degp_ref, w2_ref, b1_ref, q2_ref):
    degp = degp_ref[...]
    deg = degp[0, :N] + degp[1, :N] + 1.0
    dinv = lax.rsqrt(deg)
    agg = aggp_ref[0, :N, :F] + aggp_ref[1, :N, :F] + q1_ref[:N, :F]
    h1 = jnp.maximum(agg * dinv[:, None] + b1_ref[...][None, :], 0.0)
    p2 = jnp.dot(h1, w2_ref[...],
                 preferred_element_type=jnp.float32, precision=HIGHEST)
    q2_ref[...] = jnp.pad(p2 * dinv[:, None], ((0, NP - N), (0, FP - F)))


def _tc_c_body(aggp_ref, q2_ref, degp_ref, b2_ref, wf1_ref, bf1_ref,
               wf2_ref, bf2_ref, o_ref):
    degp = degp_ref[...]
    deg = degp[0, :N] + degp[1, :N] + 1.0
    dinv = lax.rsqrt(deg)
    agg = aggp_ref[0, :N, :F] + aggp_ref[1, :N, :F] + q2_ref[:N, :F]
    h2 = jnp.maximum(agg * dinv[:, None] + b2_ref[...][None, :], 0.0)
    pooled = jnp.sum(h2, axis=0, keepdims=True)
    o = jnp.dot(pooled, wf1_ref[...],
                preferred_element_type=jnp.float32, precision=HIGHEST)
    o = jnp.maximum(o + bf1_ref[...][None, :], 0.0)
    o = jnp.dot(o, wf2_ref[...],
                preferred_element_type=jnp.float32, precision=HIGHEST)
    o = o + bf2_ref[...][None, :]
    o = o - jnp.max(o, axis=-1, keepdims=True)
    e = jnp.exp(o)
    o_ref[...] = e / jnp.sum(e, axis=-1, keepdims=True)


def _tc_call(body, out_shape, *args):
    return pl.pallas_call(body, out_shape=out_shape)(*args)


def kernel(x, edge_index, W1, b1, W2, b2, Wf1, bf1, Wf2, bf2):
    src = edge_index[0]
    dst = edge_index[1]
    pad = jnp.arange(EP - E, dtype=jnp.int32)
    src_p = jnp.concatenate([src, pad % N]).reshape(NW, NCH, CH)
    dst_p = jnp.concatenate([dst, N + pad % (NP - N)]).reshape(NW, NCH, CH)

    zn = jnp.zeros((NP,), jnp.float32)
    znf = jnp.zeros((NP, FP), jnp.float32)
    ones = jnp.ones((CH,), jnp.float32)

    degp = _sc_degree(dst_p, zn, ones)
    q1 = _tc_call(_tc_a_body, jax.ShapeDtypeStruct((NP, FP), jnp.float32),
                  degp, x, W1)
    aggp1 = _sc_scatter(q1, src_p, dst_p, znf)
    q2 = _tc_call(_tc_b_body, jax.ShapeDtypeStruct((NP, FP), jnp.float32),
                  aggp1, q1, degp, W2, b1)
    aggp2 = _sc_scatter(q2, src_p, dst_p, znf)
    o = _tc_call(_tc_c_body, jax.ShapeDtypeStruct((1, 10), jnp.float32),
                 aggp2, q2, degp, b2, Wf1, bf1, Wf2, bf2)
    return o


# speedup vs baseline: 1.2617x; 1.2617x over previous
import functools

import jax
import jax.numpy as jnp
from jax import lax
from jax.experimental import pallas as pl
from jax.experimental.pallas import tpu as pltpu
from jax.experimental.pallas import tpu_sc as plsc

N = 10000
E = 320000
D = 128
F = 32
FP = 128

NC = 2
NS = 16
NW = NC * NS

CH = 128
NCH = 80
EPW = NCH * CH
EP = NW * EPW
NP = 10240
RT = NP // NS

_mesh = plsc.VectorSubcoreMesh(core_axis_name="c", subcore_axis_name="s")

HIGHEST = lax.Precision.HIGHEST


@functools.partial(
    pl.kernel,
    out_type=jax.ShapeDtypeStruct((NC, NP), jnp.float32),
    mesh=_mesh,
    scratch_types=[
        pltpu.VMEM((NCH, CH), jnp.int32),
        pltpu.VMEM((CH,), jnp.float32),
        pltpu.VMEM_SHARED((NP,), jnp.float32),
        pltpu.SemaphoreType.DMA,
    ],
)
def _sc_degree(dstw_hbm, zn_hbm, ones_hbm, out_hbm, dst_v, ones_v, acc_sh, sem):
    c = lax.axis_index("c")
    s = lax.axis_index("s")
    wid = s * NC + c
    r0 = s * RT
    pltpu.sync_copy(zn_hbm.at[pl.ds(r0, RT)], acc_sh.at[pl.ds(r0, RT)])
    pltpu.sync_copy(dstw_hbm.at[wid], dst_v)
    pltpu.sync_copy(ones_hbm, ones_v)
    plsc.subcore_barrier()

    def body(j, carry):
        pltpu.sync_copy(ones_v, acc_sh.at[dst_v.at[j]], add=True)
        return carry

    lax.fori_loop(0, NCH, body, 0, unroll=False)
    plsc.subcore_barrier()
    pltpu.sync_copy(acc_sh.at[pl.ds(r0, RT)], out_hbm.at[c, pl.ds(r0, RT)])


@functools.partial(
    pl.kernel,
    out_type=jax.ShapeDtypeStruct((NC, NP, FP), jnp.float32),
    mesh=_mesh,
    scratch_types=[
        pltpu.VMEM((NCH, CH), jnp.int32),
        pltpu.VMEM((CH,), jnp.int32),
        pltpu.VMEM((CH,), jnp.int32),
        pltpu.VMEM((CH, FP), jnp.float32),
        pltpu.VMEM((CH, FP), jnp.float32),
        pltpu.VMEM_SHARED((NP, FP), jnp.float32),
        pltpu.SemaphoreType.DMA,
        pltpu.SemaphoreType.DMA,
    ],
)
def _sc_scatter(q_hbm, srcw_hbm, dstw_hbm, znf_hbm, out_hbm,
                src_v, dst0, dst1, rows0, rows1, acc_sh, sem0, sem1):
    c = lax.axis_index("c")
    s = lax.axis_index("s")
    wid = s * NC + c
    r0 = s * RT
    pltpu.sync_copy(znf_hbm.at[pl.ds(r0, RT)], acc_sh.at[pl.ds(r0, RT)])
    pltpu.sync_copy(srcw_hbm.at[wid], src_v)
    plsc.subcore_barrier()

    pltpu.async_copy(q_hbm.at[src_v.at[0]], rows0, sem0)
    pltpu.async_copy(dstw_hbm.at[wid, 0], dst0, sem0)
    pltpu.async_copy(q_hbm.at[src_v.at[1]], rows1, sem1)
    pltpu.async_copy(dstw_hbm.at[wid, 1], dst1, sem1)

    def step(j, rows_v, dst_v, sem):
        pltpu.make_async_copy(q_hbm.at[src_v.at[j]], rows_v, sem).wait()
        pltpu.make_async_copy(dstw_hbm.at[wid, j], dst_v, sem).wait()
        pltpu.sync_copy(rows_v, acc_sh.at[dst_v], add=True)

    def body(i, carry):
        j = 2 * i
        step(j, rows0, dst0, sem0)
        pltpu.async_copy(q_hbm.at[src_v.at[j + 2]], rows0, sem0)
        pltpu.async_copy(dstw_hbm.at[wid, j + 2], dst0, sem0)
        step(j + 1, rows1, dst1, sem1)
        pltpu.async_copy(q_hbm.at[src_v.at[j + 3]], rows1, sem1)
        pltpu.async_copy(dstw_hbm.at[wid, j + 3], dst1, sem1)
        return carry

    lax.fori_loop(0, NCH // 2 - 1, body, 0, unroll=False)
    step(NCH - 2, rows0, dst0, sem0)
    step(NCH - 1, rows1, dst1, sem1)
    plsc.subcore_barrier()
    pltpu.sync_copy(acc_sh.at[pl.ds(r0, RT)], out_hbm.at[c, pl.ds(r0, RT)])


def _tc_mm_body(x_ref, w1_ref, p_ref):
    p_ref[...] = jnp.dot(x_ref[...], w1_ref[...],
                         preferred_element_type=jnp.float32, precision=HIGHEST)


def _tc_a_body(degp_ref, p_ref, q_ref):
    degp = degp_ref[...]
    deg = degp[0, :N] + degp[1, :N] + 1.0
    dinv = lax.rsqrt(deg)
    q_ref[...] = jnp.pad(p_ref[...] * dinv[:, None], ((0, NP - N), (0, FP - F)))


def _tc_b_body(aggp_ref, q1_ref, degp_ref, w2_ref, b1_ref, q2_ref):
    degp = degp_ref[...]
    deg = degp[0, :N] + degp[1, :N] + 1.0
    dinv = lax.rsqrt(deg)
    agg = aggp_ref[0, :N, :F] + aggp_ref[1, :N, :F] + q1_ref[:N, :F]
    h1 = jnp.maximum(agg * dinv[:, None] + b1_ref[...][None, :], 0.0)
    p2 = jnp.dot(h1, w2_ref[...],
                 preferred_element_type=jnp.float32, precision=HIGHEST)
    q2_ref[...] = jnp.pad(p2 * dinv[:, None], ((0, NP - N), (0, FP - F)))


def _tc_c_body(aggp_ref, q2_ref, degp_ref, b2_ref, wf1_ref, bf1_ref,
               wf2_ref, bf2_ref, o_ref):
    degp = degp_ref[...]
    deg = degp[0, :N] + degp[1, :N] + 1.0
    dinv = lax.rsqrt(deg)
    agg = aggp_ref[0, :N, :F] + aggp_ref[1, :N, :F] + q2_ref[:N, :F]
    h2 = jnp.maximum(agg * dinv[:, None] + b2_ref[...][None, :], 0.0)
    pooled = jnp.sum(h2, axis=0, keepdims=True)
    o = jnp.dot(pooled, wf1_ref[...],
                preferred_element_type=jnp.float32, precision=HIGHEST)
    o = jnp.maximum(o + bf1_ref[...][None, :], 0.0)
    o = jnp.dot(o, wf2_ref[...],
                preferred_element_type=jnp.float32, precision=HIGHEST)
    o = o + bf2_ref[...][None, :]
    o = o - jnp.max(o, axis=-1, keepdims=True)
    e = jnp.exp(o)
    o_ref[...] = e / jnp.sum(e, axis=-1, keepdims=True)


def _tc_call(body, out_shape, *args):
    return pl.pallas_call(body, out_shape=out_shape)(*args)


def kernel(x, edge_index, W1, b1, W2, b2, Wf1, bf1, Wf2, bf2):
    src = edge_index[0]
    dst = edge_index[1]
    pad = jnp.arange(EP - E, dtype=jnp.int32)
    src_p = jnp.concatenate([src, pad % N]).reshape(NW, NCH, CH)
    dst_p = jnp.concatenate([dst, N + pad % (NP - N)]).reshape(NW, NCH, CH)

    zn = jnp.zeros((NP,), jnp.float32)
    znf = jnp.zeros((NP, FP), jnp.float32)
    ones = jnp.ones((CH,), jnp.float32)

    degp = _sc_degree(dst_p, zn, ones)
    p1 = _tc_call(_tc_mm_body, jax.ShapeDtypeStruct((N, F), jnp.float32),
                  x, W1)
    q1 = _tc_call(_tc_a_body, jax.ShapeDtypeStruct((NP, FP), jnp.float32),
                  degp, p1)
    aggp1 = _sc_scatter(q1, src_p, dst_p, znf)
    q2 = _tc_call(_tc_b_body, jax.ShapeDtypeStruct((NP, FP), jnp.float32),
                  aggp1, q1, degp, W2, b1)
    aggp2 = _sc_scatter(q2, src_p, dst_p, znf)
    o = _tc_call(_tc_c_body, jax.ShapeDtypeStruct((1, 10), jnp.float32),
                 aggp2, q2, degp, b2, Wf1, bf1, Wf2, bf2)
    return o
